# baseline (speedup 1.0000x reference)
import functools

import jax
import jax.numpy as jnp
from jax import lax
from jax.experimental import pallas as pl
from jax.experimental.pallas import tpu as pltpu
from jax.experimental.pallas import tpu_sc as plsc

N = 1000
NP = 1024
BQ = 16384
NC, NS, L = 2, 16, 16
NW = NC * NS
BPW = BQ // NW
CHUNK = 128
NCHUNK = BPW // CHUNK


GR = 40


def _fold_body(wcr_ref, wcc_ref, wbc_ref, wab_ref, ia_ref, ib_ref, ic_ref,
               fbc_ref, fab_ref, xbc_ref, xab_ref):
    ib = ib_ref[...]
    xbc_ref[...] = ic_ref[...] * NP + ib
    xab_ref[...] = ib * NP + ia_ref[...]
    wbc = wbc_ref[...]
    m1 = jnp.max(wbc, axis=1, keepdims=True)
    lse_bc = m1 + jnp.log(jnp.sum(jnp.exp(wbc - m1), axis=1, keepdims=True))
    wab = wab_ref[...]
    m2 = jnp.max(wab, axis=1, keepdims=True)
    lse_ab = m2 + jnp.log(jnp.sum(jnp.exp(wab - m2), axis=1, keepdims=True))
    wcr = wcr_ref[...]
    mc = jnp.max(wcr)
    lse_c = mc + jnp.log(jnp.sum(jnp.exp(wcr - mc)))
    adj_bc = wbc + (wcc_ref[...] - lse_c - lse_bc)
    adj_ab = wab - lse_ab
    for k in range(GR):
        fbc_ref[pl.ds(k * NP, N)] = adj_bc[k, :]
        fab_ref[pl.ds(k * NP, N)] = adj_ab[k, :]


_fold_call = pl.pallas_call(
    _fold_body,
    grid=(N // GR,),
    in_specs=[
        pl.BlockSpec((1, N), lambda i: (0, 0)),
        pl.BlockSpec((GR, 1), lambda i: (i, 0)),
        pl.BlockSpec((GR, N), lambda i: (i, 0)),
        pl.BlockSpec((GR, N), lambda i: (i, 0)),
        pl.BlockSpec((BQ,), lambda i: (0,)),
        pl.BlockSpec((BQ,), lambda i: (0,)),
        pl.BlockSpec((BQ,), lambda i: (0,)),
    ],
    out_specs=(
        pl.BlockSpec((GR * NP,), lambda i: (i,)),
        pl.BlockSpec((GR * NP,), lambda i: (i,)),
        pl.BlockSpec((BQ,), lambda i: (0,)),
        pl.BlockSpec((BQ,), lambda i: (0,)),
    ),
    out_shape=(
        jax.ShapeDtypeStruct((N * NP,), jnp.float32),
        jax.ShapeDtypeStruct((N * NP,), jnp.float32),
        jax.ShapeDtypeStruct((BQ,), jnp.int32),
        jax.ShapeDtypeStruct((BQ,), jnp.int32),
    ),
)


@functools.cache
def _build_gather_combine():
  mesh = plsc.VectorSubcoreMesh(core_axis_name="c", subcore_axis_name="s")

  @functools.partial(
      pl.kernel,
      out_type=jax.ShapeDtypeStruct((BQ,), jnp.float32),
      mesh=mesh,
      scratch_types=[
          pltpu.VMEM((BPW,), jnp.int32),
          pltpu.VMEM((BPW,), jnp.int32),
          pltpu.VMEM((BPW,), jnp.float32),
          pltpu.VMEM((BPW,), jnp.float32),
          pltpu.VMEM((BPW,), jnp.float32),
          pltpu.SemaphoreType.DMA,
      ],
  )
  def _gather_combine(xbc_hbm, xab_hbm, fbc_hbm, fab_hbm,
                      out_hbm, xbc_v, xab_v, gbc_v, gab_v, out_v, sem):
    wid = lax.axis_index("s") * NC + lax.axis_index("c")
    base = wid * BPW
    c1 = pltpu.async_copy(xbc_hbm.at[pl.ds(base, BPW)], xbc_v, sem)
    c2 = pltpu.async_copy(xab_hbm.at[pl.ds(base, BPW)], xab_v, sem)
    c1.wait()
    c2.wait()
    g1 = pltpu.async_copy(fbc_hbm.at[xbc_v], gbc_v, sem)
    g2 = pltpu.async_copy(fab_hbm.at[xab_v], gab_v, sem)
    g1.wait()
    g2.wait()

    @pl.loop(0, BPW // L)
    def _combine(j):
        sl = pl.ds(j * L, L)
        out_v[sl] = gbc_v[sl] + gab_v[sl]

    pltpu.sync_copy(out_v, out_hbm.at[pl.ds(base, BPW)])

  return _gather_combine


def kernel(inputs, w_C, w_B_C, w_A_B):
    idx = inputs.astype(jnp.int32)
    ia, ib, ic = idx[:, 0], idx[:, 1], idx[:, 2]
    fbc, fab, xbc, xab = _fold_call(
        w_C.reshape(1, N), w_C.reshape(N, 1), w_B_C, w_A_B, ia, ib, ic)
    return _build_gather_combine()(xbc, xab, fbc, fab)


# speedup vs baseline: 1.3253x; 1.3253x over previous
import functools

import jax
import jax.numpy as jnp
from jax import lax
from jax.experimental import pallas as pl
from jax.experimental.pallas import tpu as pltpu
from jax.experimental.pallas import tpu_sc as plsc

N = 1000
NP = 1024
BQ = 16384
NC, NS, L = 2, 16, 16
NW = NC * NS
BPW = BQ // NW
CHUNK = 128
NCHUNK = BPW // CHUNK


GR = 200


def _fold_body(wcr_ref, wcc_ref, wbc_ref, wab_ref, ia_ref, ib_ref, ic_ref,
               fbc_ref, fab_ref, xbc_ref, xab_ref):
    ib = ib_ref[...]
    xbc_ref[...] = ic_ref[...] * NP + ib
    xab_ref[...] = ib * NP + ia_ref[...]
    wbc = wbc_ref[...]
    m1 = jnp.max(wbc, axis=1, keepdims=True)
    lse_bc = m1 + jnp.log(jnp.sum(jnp.exp(wbc - m1), axis=1, keepdims=True))
    wab = wab_ref[...]
    m2 = jnp.max(wab, axis=1, keepdims=True)
    lse_ab = m2 + jnp.log(jnp.sum(jnp.exp(wab - m2), axis=1, keepdims=True))
    wcr = wcr_ref[...]
    mc = jnp.max(wcr)
    lse_c = mc + jnp.log(jnp.sum(jnp.exp(wcr - mc)))
    adj_bc = wbc + (wcc_ref[...] - lse_c - lse_bc)
    adj_ab = wab - lse_ab
    for k in range(GR):
        fbc_ref[pl.ds(k * NP, N)] = adj_bc[k, :]
        fab_ref[pl.ds(k * NP, N)] = adj_ab[k, :]


_fold_call = pl.pallas_call(
    _fold_body,
    grid=(N // GR,),
    in_specs=[
        pl.BlockSpec((1, N), lambda i: (0, 0)),
        pl.BlockSpec((GR, 1), lambda i: (i, 0)),
        pl.BlockSpec((GR, N), lambda i: (i, 0)),
        pl.BlockSpec((GR, N), lambda i: (i, 0)),
        pl.BlockSpec((BQ,), lambda i: (0,)),
        pl.BlockSpec((BQ,), lambda i: (0,)),
        pl.BlockSpec((BQ,), lambda i: (0,)),
    ],
    out_specs=(
        pl.BlockSpec((GR * NP,), lambda i: (i,)),
        pl.BlockSpec((GR * NP,), lambda i: (i,)),
        pl.BlockSpec((BQ,), lambda i: (0,)),
        pl.BlockSpec((BQ,), lambda i: (0,)),
    ),
    out_shape=(
        jax.ShapeDtypeStruct((N * NP,), jnp.float32),
        jax.ShapeDtypeStruct((N * NP,), jnp.float32),
        jax.ShapeDtypeStruct((BQ,), jnp.int32),
        jax.ShapeDtypeStruct((BQ,), jnp.int32),
    ),
)


@functools.cache
def _build_gather_combine():
  mesh = plsc.VectorSubcoreMesh(core_axis_name="c", subcore_axis_name="s")

  @functools.partial(
      pl.kernel,
      out_type=jax.ShapeDtypeStruct((BQ,), jnp.float32),
      mesh=mesh,
      scratch_types=[
          pltpu.VMEM((BPW,), jnp.int32),
          pltpu.VMEM((BPW,), jnp.int32),
          pltpu.VMEM((BPW,), jnp.float32),
          pltpu.VMEM((BPW,), jnp.float32),
          pltpu.SemaphoreType.DMA,
      ],
  )
  def _gather_combine(xbc_hbm, xab_hbm, fbc_hbm, fab_hbm,
                      out_hbm, xbc_v, xab_v, gbc_v, gab_v, sem):
    wid = lax.axis_index("s") * NC + lax.axis_index("c")
    base = wid * BPW
    c1 = pltpu.async_copy(xbc_hbm.at[pl.ds(base, BPW)], xbc_v, sem)
    c2 = pltpu.async_copy(xab_hbm.at[pl.ds(base, BPW)], xab_v, sem)
    c1.wait()
    g1 = pltpu.async_copy(fbc_hbm.at[xbc_v], gbc_v, sem)
    c2.wait()
    g2 = pltpu.async_copy(fab_hbm.at[xab_v], gab_v, sem)
    g1.wait()
    g2.wait()

    @pl.loop(0, BPW // L)
    def _combine(j):
        sl = pl.ds(j * L, L)
        gbc_v[sl] = gbc_v[sl] + gab_v[sl]

    pltpu.sync_copy(gbc_v, out_hbm.at[pl.ds(base, BPW)])

  return _gather_combine


def kernel(inputs, w_C, w_B_C, w_A_B):
    idx = inputs.astype(jnp.int32)
    ia, ib, ic = idx[:, 0], idx[:, 1], idx[:, 2]
    fbc, fab, xbc, xab = _fold_call(
        w_C.reshape(1, N), w_C.reshape(N, 1), w_B_C, w_A_B, ia, ib, ic)
    return _build_gather_combine()(xbc, xab, fbc, fab)


# speedup vs baseline: 1.3433x; 1.0136x over previous
import functools

import jax
import jax.numpy as jnp
from jax import lax
from jax.experimental import pallas as pl
from jax.experimental.pallas import tpu as pltpu
from jax.experimental.pallas import tpu_sc as plsc

N = 1000
NP = 1024
BQ = 16384
NC, NS, L = 2, 16, 16
NW = NC * NS
BPW = BQ // NW
CHUNK = 128
NCHUNK = BPW // CHUNK


GR = 200


def _fold_body(wcr_ref, wcc_ref, wbc_ref, wab_ref, ia_ref, ib_ref, ic_ref,
               fbc_ref, fab_ref, xbc_ref, xab_ref):
    ib = ib_ref[...]
    xbc_ref[...] = ic_ref[...] * NP + ib
    xab_ref[...] = ib * NP + ia_ref[...]
    wbc = wbc_ref[...]
    m1 = jnp.max(wbc, axis=1, keepdims=True)
    lse_bc = m1 + jnp.log(jnp.sum(jnp.exp(wbc - m1), axis=1, keepdims=True))
    wab = wab_ref[...]
    m2 = jnp.max(wab, axis=1, keepdims=True)
    lse_ab = m2 + jnp.log(jnp.sum(jnp.exp(wab - m2), axis=1, keepdims=True))
    wcr = wcr_ref[...]
    mc = jnp.max(wcr)
    lse_c = mc + jnp.log(jnp.sum(jnp.exp(wcr - mc)))
    adj_bc = wbc + (wcc_ref[...] - lse_c - lse_bc)
    adj_ab = wab - lse_ab
    for k in range(GR):
        fbc_ref[pl.ds(k * NP, N)] = adj_bc[k, :]
        fab_ref[pl.ds(k * NP, N)] = adj_ab[k, :]


_fold_call = pl.pallas_call(
    _fold_body,
    grid=(N // GR,),
    in_specs=[
        pl.BlockSpec((1, N), lambda i: (0, 0)),
        pl.BlockSpec((GR, 1), lambda i: (i, 0)),
        pl.BlockSpec((GR, N), lambda i: (i, 0)),
        pl.BlockSpec((GR, N), lambda i: (i, 0)),
        pl.BlockSpec((BQ,), lambda i: (0,)),
        pl.BlockSpec((BQ,), lambda i: (0,)),
        pl.BlockSpec((BQ,), lambda i: (0,)),
    ],
    out_specs=(
        pl.BlockSpec((GR * NP,), lambda i: (i,)),
        pl.BlockSpec((GR * NP,), lambda i: (i,)),
        pl.BlockSpec((BQ,), lambda i: (0,)),
        pl.BlockSpec((BQ,), lambda i: (0,)),
    ),
    out_shape=(
        jax.ShapeDtypeStruct((N * NP,), jnp.float32),
        jax.ShapeDtypeStruct((N * NP,), jnp.float32),
        jax.ShapeDtypeStruct((BQ,), jnp.int32),
        jax.ShapeDtypeStruct((BQ,), jnp.int32),
    ),
)


@functools.cache
def _build_gather_combine():
  mesh = plsc.VectorSubcoreMesh(core_axis_name="c", subcore_axis_name="s")

  @functools.partial(
      pl.kernel,
      out_type=jax.ShapeDtypeStruct((BQ,), jnp.float32),
      mesh=mesh,
      scratch_types=[
          pltpu.VMEM((BPW,), jnp.int32),
          pltpu.VMEM((BPW,), jnp.int32),
          pltpu.VMEM((BPW,), jnp.float32),
          pltpu.VMEM((BPW,), jnp.float32),
          pltpu.VMEM((BPW,), jnp.float32),
          pltpu.SemaphoreType.DMA,
      ],
  )
  def _gather_combine(xbc_hbm, xab_hbm, fbc_hbm, fab_hbm,
                      out_hbm, xbc_v, xab_v, gbc_v, gab_v, out_v, sem):
    wid = lax.axis_index("s") * NC + lax.axis_index("c")
    base = wid * BPW
    c1 = pltpu.async_copy(xbc_hbm.at[pl.ds(base, BPW)], xbc_v, sem)
    c2 = pltpu.async_copy(xab_hbm.at[pl.ds(base, BPW)], xab_v, sem)
    c1.wait()
    c2.wait()
    g1 = pltpu.async_copy(fbc_hbm.at[xbc_v], gbc_v, sem)
    g2 = pltpu.async_copy(fab_hbm.at[xab_v], gab_v, sem)
    g1.wait()
    g2.wait()

    @pl.loop(0, BPW // L)
    def _combine(j):
        sl = pl.ds(j * L, L)
        out_v[sl] = gbc_v[sl] + gab_v[sl]

    pltpu.sync_copy(out_v, out_hbm.at[pl.ds(base, BPW)])

  return _gather_combine


def kernel(inputs, w_C, w_B_C, w_A_B):
    idx = inputs.astype(jnp.int32)
    ia, ib, ic = idx[:, 0], idx[:, 1], idx[:, 2]
    fbc, fab, xbc, xab = _fold_call(
        w_C.reshape(1, N), w_C.reshape(N, 1), w_B_C, w_A_B, ia, ib, ic)
    return _build_gather_combine()(xbc, xab, fbc, fab)


# speedup vs baseline: 1.3458x; 1.0019x over previous
import functools

import jax
import jax.numpy as jnp
from jax import lax
from jax.experimental import pallas as pl
from jax.experimental.pallas import tpu as pltpu
from jax.experimental.pallas import tpu_sc as plsc

N = 1000
NP = 1024
BQ = 16384
NC, NS, L = 2, 16, 16
NW = NC * NS
BPW = BQ // NW


GR = 200


def _fold_body(wcr_ref, wcc_ref, wbc_ref, wab_ref, ia_ref, ib_ref, ic_ref,
               fbc_ref, fab_ref, xbc_ref, xab_ref):
    ib = ib_ref[...]
    xbc_ref[...] = ic_ref[...] * NP + ib
    xab_ref[...] = ib * NP + ia_ref[...]
    wbc = wbc_ref[...]
    m1 = jnp.max(wbc, axis=1, keepdims=True)
    lse_bc = m1 + jnp.log(jnp.sum(jnp.exp(wbc - m1), axis=1, keepdims=True))
    wab = wab_ref[...]
    m2 = jnp.max(wab, axis=1, keepdims=True)
    lse_ab = m2 + jnp.log(jnp.sum(jnp.exp(wab - m2), axis=1, keepdims=True))
    wcr = wcr_ref[...]
    mc = jnp.max(wcr)
    lse_c = mc + jnp.log(jnp.sum(jnp.exp(wcr - mc)))
    adj_bc = wbc + (wcc_ref[...] - lse_c - lse_bc)
    adj_ab = wab - lse_ab
    for k in range(GR):
        fbc_ref[pl.ds(k * NP, N)] = adj_bc[k, :]
        fab_ref[pl.ds(k * NP, N)] = adj_ab[k, :]


_fold_call = pl.pallas_call(
    _fold_body,
    grid=(N // GR,),
    in_specs=[
        pl.BlockSpec((1, N), lambda i: (0, 0)),
        pl.BlockSpec((GR, 1), lambda i: (i, 0)),
        pl.BlockSpec((GR, N), lambda i: (i, 0)),
        pl.BlockSpec((GR, N), lambda i: (i, 0)),
        pl.BlockSpec((BQ,), lambda i: (0,)),
        pl.BlockSpec((BQ,), lambda i: (0,)),
        pl.BlockSpec((BQ,), lambda i: (0,)),
    ],
    out_specs=(
        pl.BlockSpec((GR * NP,), lambda i: (i,)),
        pl.BlockSpec((GR * NP,), lambda i: (i,)),
        pl.BlockSpec((BQ,), lambda i: (0,)),
        pl.BlockSpec((BQ,), lambda i: (0,)),
    ),
    out_shape=(
        jax.ShapeDtypeStruct((N * NP,), jnp.float32),
        jax.ShapeDtypeStruct((N * NP,), jnp.float32),
        jax.ShapeDtypeStruct((BQ,), jnp.int32),
        jax.ShapeDtypeStruct((BQ,), jnp.int32),
    ),
)


@functools.cache
def _build_gather_combine():
  mesh = plsc.VectorSubcoreMesh(core_axis_name="c", subcore_axis_name="s")

  @functools.partial(
      pl.kernel,
      out_type=jax.ShapeDtypeStruct((BQ,), jnp.float32),
      mesh=mesh,
      scratch_types=[
          pltpu.VMEM((BPW,), jnp.int32),
          pltpu.VMEM((BPW,), jnp.int32),
          pltpu.VMEM((BPW,), jnp.float32),
          pltpu.VMEM((BPW,), jnp.float32),
          pltpu.VMEM((BPW,), jnp.float32),
          pltpu.SemaphoreType.DMA,
      ],
  )
  def _gather_combine(xbc_hbm, xab_hbm, fbc_hbm, fab_hbm,
                      out_hbm, xbc_v, xab_v, gbc_v, gab_v, out_v, sem):
    wid = lax.axis_index("s") * NC + lax.axis_index("c")
    base = wid * BPW
    c1 = pltpu.async_copy(xbc_hbm.at[pl.ds(base, BPW)], xbc_v, sem)
    c2 = pltpu.async_copy(xab_hbm.at[pl.ds(base, BPW)], xab_v, sem)
    c1.wait()
    c2.wait()
    g1 = pltpu.async_copy(fbc_hbm.at[xbc_v], gbc_v, sem)
    g2 = pltpu.async_copy(fab_hbm.at[xab_v], gab_v, sem)
    g1.wait()
    g2.wait()

    @pl.loop(0, BPW // L)
    def _combine(j):
        sl = pl.ds(j * L, L)
        out_v[sl] = gbc_v[sl] + gab_v[sl]

    pltpu.sync_copy(out_v, out_hbm.at[pl.ds(base, BPW)])

  return _gather_combine


def kernel(inputs, w_C, w_B_C, w_A_B):
    idx = inputs.astype(jnp.int32)
    ia, ib, ic = idx[:, 0], idx[:, 1], idx[:, 2]
    fbc, fab, xbc, xab = _fold_call(
        w_C.reshape(1, N), w_C.reshape(N, 1), w_B_C, w_A_B, ia, ib, ic)
    return _build_gather_combine()(xbc, xab, fbc, fab)
